# Initial kernel scaffold; baseline (speedup 1.0000x reference)
import jax
import jax.numpy as jnp
from jax.experimental import pallas as pl


def kernel(src_node_feat, tgt_node_feat, src_node_coord, tgt_node_coord, edge_list, edge_attr, W_es2t0, b_es2t0, W_es2t1, b_es2t1, W_et2s0, b_et2s0, W_et2s1, b_et2s1, W_nt0, b_nt0, W_nt1, b_nt1, W_ns0, b_ns0, W_ns1, b_ns1):
    raise NotImplementedError("write your pallas kernel here")


# speedup vs baseline: 2.3839x; 2.3839x over previous
import functools

import jax
import jax.numpy as jnp
from jax import lax
from jax.experimental import pallas as pl
from jax.experimental.pallas import tpu as pltpu
from jax.experimental.pallas import tpu_sc as plsc

N = 10000
E = 320000
D = 128
H = 128
EA = 16
TW = 144

NC = 2
NS = 16
NW = NC * NS

EPW = E // NW
GC = 80
GNCH = EPW // GC

_sc_mesh = plsc.VectorSubcoreMesh(core_axis_name="c", subcore_axis_name="s")
_sc_params = pltpu.CompilerParams(use_tc_tiling_on_sc=False)


@functools.partial(
    pl.kernel,
    out_type=(
        jax.ShapeDtypeStruct((E, TW), jnp.float32),
        jax.ShapeDtypeStruct((E, TW), jnp.float32),
    ),
    mesh=_sc_mesh,
    scratch_types=[
        pltpu.VMEM((GC,), jnp.int32),
        pltpu.VMEM((GC, TW), jnp.float32),
        pltpu.VMEM((GC,), jnp.int32),
        pltpu.VMEM((GC, TW), jnp.float32),
        pltpu.SemaphoreType.DMA,
        pltpu.SemaphoreType.DMA,
    ],
    compiler_params=_sc_params,
)
def _gather_k(tsrc_hbm, ttgt_hbm, esrc_hbm, etgt_hbm, gsrc_hbm, gtgt_hbm,
              idx_a, rows_a, idx_b, rows_b, sem_a, sem_b):
    c = lax.axis_index("c")
    s = lax.axis_index("s")
    wid = s * NC + c
    base = pl.multiple_of(wid * EPW, 8)

    def body(j, _):
        off = pl.multiple_of(base + j * GC, 8)
        pltpu.sync_copy(esrc_hbm.at[pl.ds(off, GC)], idx_a)
        pltpu.sync_copy(etgt_hbm.at[pl.ds(off, GC)], idx_b)
        cp_a = pltpu.async_copy(tsrc_hbm.at[idx_a], rows_a, sem_a)
        cp_b = pltpu.async_copy(ttgt_hbm.at[idx_b], rows_b, sem_b)
        cp_a.wait()
        pltpu.sync_copy(rows_a, gsrc_hbm.at[pl.ds(off, GC)])
        cp_b.wait()
        pltpu.sync_copy(rows_b, gtgt_hbm.at[pl.ds(off, GC)])
        return _

    lax.fori_loop(0, GNCH, body, 0)


EB = 2000


def _edge_body(gsrc, gtgt, ea,
               w1s, w1t, w1r, w1a, b10, w11, b11,
               w2s, w2t, w2r, w2a, b20, w21, b21,
               h1o, h2o):
    src = gsrc[:, :D]
    tgtf = gtgt[:, :D]
    dd = gtgt[:, D:TW] - gsrc[:, D:TW]
    radial = jnp.sum(dd * dd, axis=1, keepdims=True)
    eab = ea[...]

    def mlp(ws, wt, wr, wa, b0, w1, b1):
        u = jnp.dot(src, ws[...], preferred_element_type=jnp.float32)
        u = u + jnp.dot(tgtf, wt[...], preferred_element_type=jnp.float32)
        u = u + jnp.dot(eab, wa[...], preferred_element_type=jnp.float32)
        u = u + radial * wr[...]
        u = u + b0[...]
        z = jnp.maximum(u, 0.0)
        h = jnp.dot(z, w1[...], preferred_element_type=jnp.float32) + b1[...]
        return jnp.maximum(h, 0.0)

    h1o[...] = mlp(w1s, w1t, w1r, w1a, b10, w11, b11)
    h2o[...] = mlp(w2s, w2t, w2r, w2a, b20, w21, b21)


def _full(shape):
    return pl.BlockSpec(shape, lambda i: (0, 0))


_edge_call = pl.pallas_call(
    _edge_body,
    grid=(E // EB,),
    in_specs=[
        pl.BlockSpec((EB, TW), lambda i: (i, 0)),
        pl.BlockSpec((EB, TW), lambda i: (i, 0)),
        pl.BlockSpec((EB, EA), lambda i: (i, 0)),
        _full((D, H)), _full((D, H)), _full((1, H)), _full((EA, H)),
        _full((1, H)), _full((H, H)), _full((1, H)),
        _full((D, H)), _full((D, H)), _full((1, H)), _full((EA, H)),
        _full((1, H)), _full((H, H)), _full((1, H)),
    ],
    out_specs=[
        pl.BlockSpec((EB, H), lambda i: (i, 0)),
        pl.BlockSpec((EB, H), lambda i: (i, 0)),
    ],
    out_shape=[
        jax.ShapeDtypeStruct((E, H), jnp.float32),
        jax.ShapeDtypeStruct((E, H), jnp.float32),
    ],
)


EPT = E // NS
SC_C = 80
SNCH = EPT // SC_C
NPT = N // NS


@functools.partial(
    pl.kernel,
    out_type=(
        jax.ShapeDtypeStruct((N, H), jnp.float32),
        jax.ShapeDtypeStruct((N, H), jnp.float32),
    ),
    mesh=_sc_mesh,
    scratch_types=[
        pltpu.VMEM((SC_C,), jnp.int32),
        pltpu.VMEM((SC_C, H), jnp.float32),
        pltpu.VMEM_SHARED((N, H), jnp.float32),
    ],
    compiler_params=_sc_params,
)
def _scatter_k(h1_hbm, h2_hbm, etgt_hbm, esrc_hbm, zeros_hbm,
               agg1_hbm, agg2_hbm, idx_v, rows_v, acc_sh):
    c = lax.axis_index("c")
    s = lax.axis_index("s")
    nbase = pl.multiple_of(s * NPT, 8)
    pltpu.sync_copy(zeros_hbm, acc_sh.at[pl.ds(nbase, NPT)])
    plsc.subcore_barrier()

    def do(h_hbm, eidx_hbm):
        base = pl.multiple_of(s * EPT, 8)

        def body(j, _):
            off = pl.multiple_of(base + j * SC_C, 8)
            pltpu.sync_copy(eidx_hbm.at[pl.ds(off, SC_C)], idx_v)
            pltpu.sync_copy(h_hbm.at[pl.ds(off, SC_C)], rows_v)
            pltpu.sync_copy(rows_v, acc_sh.at[idx_v], add=True)
            return _

        lax.fori_loop(0, SNCH, body, 0)

    @pl.when(c == 0)
    def _():
        do(h1_hbm, etgt_hbm)

    @pl.when(c == 1)
    def _():
        do(h2_hbm, esrc_hbm)

    plsc.subcore_barrier()

    @pl.when(c == 0)
    def _():
        pltpu.sync_copy(acc_sh.at[pl.ds(nbase, NPT)],
                        agg1_hbm.at[pl.ds(nbase, NPT)])

    @pl.when(c == 1)
    def _():
        pltpu.sync_copy(acc_sh.at[pl.ds(nbase, NPT)],
                        agg2_hbm.at[pl.ds(nbase, NPT)])


NB = 2000


def _node_body(tf, a1, sf, a2,
               wtf, wta, bt0, wt1, bt1,
               wsf, wsa, bs0, ws1, bs1,
               tgt_o, src_o):
    def upd(x, a, wf, wa, b0, w1, b1):
        u = jnp.dot(x, wf[...], preferred_element_type=jnp.float32)
        u = u + jnp.dot(a, wa[...], preferred_element_type=jnp.float32)
        u = u + b0[...]
        z = jnp.maximum(u, 0.0)
        return x + jnp.dot(z, w1[...], preferred_element_type=jnp.float32) + b1[...]

    tgt_o[...] = upd(tf[...], a1[...], wtf, wta, bt0, wt1, bt1)
    src_o[...] = upd(sf[...], a2[...], wsf, wsa, bs0, ws1, bs1)


_node_call = pl.pallas_call(
    _node_body,
    grid=(N // NB,),
    in_specs=[
        pl.BlockSpec((NB, D), lambda i: (i, 0)),
        pl.BlockSpec((NB, H), lambda i: (i, 0)),
        pl.BlockSpec((NB, D), lambda i: (i, 0)),
        pl.BlockSpec((NB, H), lambda i: (i, 0)),
        _full((D, H)), _full((H, H)), _full((1, H)), _full((H, H)), _full((1, H)),
        _full((D, H)), _full((H, H)), _full((1, H)), _full((H, H)), _full((1, H)),
    ],
    out_specs=[
        pl.BlockSpec((NB, D), lambda i: (i, 0)),
        pl.BlockSpec((NB, D), lambda i: (i, 0)),
    ],
    out_shape=[
        jax.ShapeDtypeStruct((N, D), jnp.float32),
        jax.ShapeDtypeStruct((N, D), jnp.float32),
    ],
)


def kernel(src_node_feat, tgt_node_feat, src_node_coord, tgt_node_coord,
           edge_list, edge_attr,
           W_es2t0, b_es2t0, W_es2t1, b_es2t1,
           W_et2s0, b_et2s0, W_et2s1, b_et2s1,
           W_nt0, b_nt0, W_nt1, b_nt1,
           W_ns0, b_ns0, W_ns1, b_ns1):
    f32 = jnp.float32
    edge_src = edge_list[0]
    edge_tgt = edge_list[1]

    tsrc = jnp.concatenate(
        [src_node_feat, jnp.pad(src_node_coord, ((0, 0), (0, TW - D - 3)))], axis=1)
    ttgt = jnp.concatenate(
        [tgt_node_feat, jnp.pad(tgt_node_coord, ((0, 0), (0, TW - D - 3)))], axis=1)

    gsrc, gtgt = _gather_k(tsrc, ttgt, edge_src, edge_tgt)

    def esplit(W):
        return (W[:, :D].T, W[:, D:2 * D].T, W[:, 2 * D].reshape(1, H),
                W[:, 2 * D + 1:].T)

    w1s, w1t, w1r, w1a = esplit(W_es2t0)
    w2s, w2t, w2r, w2a = esplit(W_et2s0)

    h1, h2 = _edge_call(
        gsrc, gtgt, edge_attr,
        w1s, w1t, w1r, w1a, b_es2t0.reshape(1, H), W_es2t1.T, b_es2t1.reshape(1, H),
        w2s, w2t, w2r, w2a, b_et2s0.reshape(1, H), W_et2s1.T, b_et2s1.reshape(1, H),
    )

    zeros = jnp.zeros((NPT, H), f32)
    agg1, agg2 = _scatter_k(h1, h2, edge_tgt, edge_src, zeros)

    tgt_out, src_out = _node_call(
        tgt_node_feat, agg1, src_node_feat, agg2,
        W_nt0[:, :D].T, W_nt0[:, D:].T, b_nt0.reshape(1, H), W_nt1.T, b_nt1.reshape(1, H),
        W_ns0[:, :D].T, W_ns0[:, D:].T, b_ns0.reshape(1, H), W_ns1.T, b_ns1.reshape(1, H),
    )
    return (tgt_out, src_out)


# speedup vs baseline: 2.6738x; 1.1216x over previous
import functools

import jax
import jax.numpy as jnp
from jax import lax
from jax.experimental import pallas as pl
from jax.experimental.pallas import tpu as pltpu
from jax.experimental.pallas import tpu_sc as plsc

N = 10000
E = 320000
D = 128
H = 128
EA = 16
CW = 16

NC = 2
NS = 16
NW = NC * NS

_sc_mesh = plsc.VectorSubcoreMesh(core_axis_name="c", subcore_axis_name="s")
_sc_params = pltpu.CompilerParams(use_tc_tiling_on_sc=False)

EPW = E // NW
GC = 80
GNCH = EPW // GC
GR = 5
GNG = GNCH // GR


@functools.partial(
    pl.kernel,
    out_type=(
        jax.ShapeDtypeStruct((E, D), jnp.bfloat16),
        jax.ShapeDtypeStruct((E, D), jnp.bfloat16),
        jax.ShapeDtypeStruct((E, CW), jnp.float32),
        jax.ShapeDtypeStruct((E, CW), jnp.float32),
    ),
    mesh=_sc_mesh,
    scratch_types=[
        pltpu.VMEM((2, EPW), jnp.int32),
        [pltpu.VMEM((GC, D), jnp.bfloat16) for _ in range(GR)],
        [pltpu.VMEM((GC, D), jnp.bfloat16) for _ in range(GR)],
        [pltpu.VMEM((GC, CW), jnp.float32) for _ in range(GR)],
        [pltpu.VMEM((GC, CW), jnp.float32) for _ in range(GR)],
        [pltpu.SemaphoreType.DMA for _ in range(GR)],
        [pltpu.SemaphoreType.DMA for _ in range(GR)],
    ],
    compiler_params=_sc_params,
)
def _gather_k(tsrc_hbm, ttgt_hbm, csrc_hbm, ctgt_hbm, elist_hbm,
              gsf_hbm, gtf_hbm, gsc_hbm, gtc_hbm,
              idx_all, sfeat, tfeat, scrd, tcrd, gsems, wsems):
    c = lax.axis_index("c")
    s = lax.axis_index("s")
    wid = s * NC + c
    base = pl.multiple_of(wid * EPW, 8)
    pltpu.sync_copy(elist_hbm.at[:, pl.ds(base, EPW)], idx_all)

    def pairs(b):
        return ((tsrc_hbm, sfeat[b], 0), (ttgt_hbm, tfeat[b], 1),
                (csrc_hbm, scrd[b], 0), (ctgt_hbm, tcrd[b], 1))

    def start_gathers(b, cof):
        for tab, buf, which in pairs(b):
            idx = idx_all.at[which, pl.ds(cof, GC)]
            pltpu.async_copy(tab.at[idx], buf, gsems[b])

    def wait_gathers(b, cof):
        for tab, buf, which in pairs(b):
            idx = idx_all.at[which, pl.ds(cof, GC)]
            pltpu.make_async_copy(tab.at[idx], buf, gsems[b]).wait()

    def outs(b, goff):
        return ((sfeat[b], gsf_hbm), (tfeat[b], gtf_hbm),
                (scrd[b], gsc_hbm), (tcrd[b], gtc_hbm))

    for b in range(GR):
        start_gathers(b, b * GC)

    def body(g, carry):
        wdescs = []
        for b in range(GR):
            cof = pl.multiple_of(g * (GR * GC) + b * GC, 8)
            goff = pl.multiple_of(base + cof, 8)
            wait_gathers(b, cof)
            slot = []
            for buf, out in outs(b, goff):
                slot.append(pltpu.async_copy(buf, out.at[pl.ds(goff, GC)], wsems[b]))
            wdescs.append(slot)
        for b in range(GR):
            for d in wdescs[b]:
                d.wait()

            @pl.when(g < GNG - 1)
            def _(b=b):
                ncof = pl.multiple_of((g + 1) * (GR * GC) + b * GC, 8)
                start_gathers(b, ncof)
        return carry

    lax.fori_loop(0, GNG, body, 0)


EB = 2000


def _edge_body(gsf, gtf, gsc, gtc, ea,
               w1s, w1t, w1r, w1a, b10, w11, b11,
               w2s, w2t, w2r, w2a, b20, w21, b21,
               h1o, h2o):
    dd = gtc[...] - gsc[...]
    radial = jnp.sum(dd * dd, axis=1, keepdims=True)
    src = gsf[...]
    tgtf = gtf[...]
    eab = ea[...]

    def mlp(ws, wt, wr, wa, b0, w1, b1):
        u = jnp.dot(src, ws[...], preferred_element_type=jnp.float32)
        u = u + jnp.dot(tgtf, wt[...], preferred_element_type=jnp.float32)
        u = u + jnp.dot(eab, wa[...], preferred_element_type=jnp.float32)
        u = u + radial * wr[...]
        u = u + b0[...]
        z = jnp.maximum(u, 0.0).astype(jnp.bfloat16)
        h = jnp.dot(z, w1[...], preferred_element_type=jnp.float32) + b1[...]
        return jnp.maximum(h, 0.0)

    h1o[...] = mlp(w1s, w1t, w1r, w1a, b10, w11, b11)
    h2o[...] = mlp(w2s, w2t, w2r, w2a, b20, w21, b21)


def _full(shape):
    return pl.BlockSpec(shape, lambda i: (0, 0))


_edge_call = pl.pallas_call(
    _edge_body,
    grid=(E // EB,),
    in_specs=[
        pl.BlockSpec((EB, D), lambda i: (i, 0)),
        pl.BlockSpec((EB, D), lambda i: (i, 0)),
        pl.BlockSpec((EB, CW), lambda i: (i, 0)),
        pl.BlockSpec((EB, CW), lambda i: (i, 0)),
        pl.BlockSpec((EB, EA), lambda i: (i, 0)),
        _full((D, H)), _full((D, H)), _full((1, H)), _full((EA, H)),
        _full((1, H)), _full((H, H)), _full((1, H)),
        _full((D, H)), _full((D, H)), _full((1, H)), _full((EA, H)),
        _full((1, H)), _full((H, H)), _full((1, H)),
    ],
    out_specs=[
        pl.BlockSpec((EB, H), lambda i: (i, 0)),
        pl.BlockSpec((EB, H), lambda i: (i, 0)),
    ],
    out_shape=[
        jax.ShapeDtypeStruct((E, H), jnp.float32),
        jax.ShapeDtypeStruct((E, H), jnp.float32),
    ],
)


EPT = E // NS
SC_C = 80
SNCH = EPT // SC_C
SR = 2
SNG = SNCH // SR
NPT = N // NS


@functools.partial(
    pl.kernel,
    out_type=(
        jax.ShapeDtypeStruct((N, H), jnp.float32),
        jax.ShapeDtypeStruct((N, H), jnp.float32),
    ),
    mesh=_sc_mesh,
    scratch_types=[
        pltpu.VMEM((SNCH, SC_C), jnp.int32),
        [pltpu.VMEM((SC_C, H), jnp.float32) for _ in range(SR)],
        pltpu.VMEM_SHARED((N, H), jnp.float32),
        [pltpu.SemaphoreType.DMA for _ in range(SR)],
        [pltpu.SemaphoreType.DMA for _ in range(SR)],
    ],
    compiler_params=_sc_params,
)
def _scatter_k(h1_hbm, h2_hbm, etgt_hbm, esrc_hbm, zeros_hbm,
               agg1_hbm, agg2_hbm, idxm, rows, acc_sh, lsems, ssems):
    c = lax.axis_index("c")
    s = lax.axis_index("s")
    nbase = pl.multiple_of(s * NPT, 8)
    pltpu.sync_copy(zeros_hbm, acc_sh.at[pl.ds(nbase, NPT)])

    @pl.when(c == 0)
    def _():
        pltpu.sync_copy(etgt_hbm.at[s], idxm)

    @pl.when(c == 1)
    def _():
        pltpu.sync_copy(esrc_hbm.at[s], idxm)

    plsc.subcore_barrier()

    def run(h_hbm):
        base = pl.multiple_of(s * EPT, 8)

        def start_load(b, j):
            off = pl.multiple_of(base + j * SC_C, 8)
            pltpu.async_copy(h_hbm.at[pl.ds(off, SC_C)], rows[b], lsems[b])

        for b in range(SR):
            start_load(b, b)

        def body(g, carry):
            sdescs = []
            for b in range(SR):
                j = g * SR + b
                off = pl.multiple_of(base + j * SC_C, 8)
                pltpu.make_async_copy(
                    h_hbm.at[pl.ds(off, SC_C)], rows[b], lsems[b]).wait()
                sdescs.append(pltpu.async_copy(
                    rows[b], acc_sh.at[idxm.at[j]], ssems[b], add=True))
            for b in range(SR):
                sdescs[b].wait()

                @pl.when(g < SNG - 1)
                def _(b=b):
                    start_load(b, (g + 1) * SR + b)
            return carry

        lax.fori_loop(0, SNG, body, 0)

    @pl.when(c == 0)
    def _():
        run(h1_hbm)

    @pl.when(c == 1)
    def _():
        run(h2_hbm)

    plsc.subcore_barrier()

    @pl.when(c == 0)
    def _():
        pltpu.sync_copy(acc_sh.at[pl.ds(nbase, NPT)],
                        agg1_hbm.at[pl.ds(nbase, NPT)])

    @pl.when(c == 1)
    def _():
        pltpu.sync_copy(acc_sh.at[pl.ds(nbase, NPT)],
                        agg2_hbm.at[pl.ds(nbase, NPT)])


NB = 2000


def _node_body(tf, a1, sf, a2,
               wtf, wta, bt0, wt1, bt1,
               wsf, wsa, bs0, ws1, bs1,
               tgt_o, src_o):
    def upd(x, a, wf, wa, b0, w1, b1):
        xb = x.astype(jnp.bfloat16)
        ab = a.astype(jnp.bfloat16)
        u = jnp.dot(xb, wf[...], preferred_element_type=jnp.float32)
        u = u + jnp.dot(ab, wa[...], preferred_element_type=jnp.float32)
        u = u + b0[...]
        z = jnp.maximum(u, 0.0).astype(jnp.bfloat16)
        return x + jnp.dot(z, w1[...], preferred_element_type=jnp.float32) + b1[...]

    tgt_o[...] = upd(tf[...], a1[...], wtf, wta, bt0, wt1, bt1)
    src_o[...] = upd(sf[...], a2[...], wsf, wsa, bs0, ws1, bs1)


_node_call = pl.pallas_call(
    _node_body,
    grid=(N // NB,),
    in_specs=[
        pl.BlockSpec((NB, D), lambda i: (i, 0)),
        pl.BlockSpec((NB, H), lambda i: (i, 0)),
        pl.BlockSpec((NB, D), lambda i: (i, 0)),
        pl.BlockSpec((NB, H), lambda i: (i, 0)),
        _full((D, H)), _full((H, H)), _full((1, H)), _full((H, H)), _full((1, H)),
        _full((D, H)), _full((H, H)), _full((1, H)), _full((H, H)), _full((1, H)),
    ],
    out_specs=[
        pl.BlockSpec((NB, D), lambda i: (i, 0)),
        pl.BlockSpec((NB, D), lambda i: (i, 0)),
    ],
    out_shape=[
        jax.ShapeDtypeStruct((N, D), jnp.float32),
        jax.ShapeDtypeStruct((N, D), jnp.float32),
    ],
)


def kernel(src_node_feat, tgt_node_feat, src_node_coord, tgt_node_coord,
           edge_list, edge_attr,
           W_es2t0, b_es2t0, W_es2t1, b_es2t1,
           W_et2s0, b_et2s0, W_et2s1, b_et2s1,
           W_nt0, b_nt0, W_nt1, b_nt1,
           W_ns0, b_ns0, W_ns1, b_ns1):
    f32 = jnp.float32
    bf16 = jnp.bfloat16

    tsrc = src_node_feat.astype(bf16)
    ttgt = tgt_node_feat.astype(bf16)
    csrc = jnp.pad(src_node_coord, ((0, 0), (0, CW - 3)))
    ctgt = jnp.pad(tgt_node_coord, ((0, 0), (0, CW - 3)))

    gsf, gtf, gsc, gtc = _gather_k(tsrc, ttgt, csrc, ctgt, edge_list)

    def esplit(W):
        return (W[:, :D].T.astype(bf16), W[:, D:2 * D].T.astype(bf16),
                W[:, 2 * D].reshape(1, H), W[:, 2 * D + 1:].T.astype(bf16))

    w1s, w1t, w1r, w1a = esplit(W_es2t0)
    w2s, w2t, w2r, w2a = esplit(W_et2s0)

    h1, h2 = _edge_call(
        gsf, gtf, gsc, gtc, edge_attr.astype(bf16),
        w1s, w1t, w1r, w1a, b_es2t0.reshape(1, H),
        W_es2t1.T.astype(bf16), b_es2t1.reshape(1, H),
        w2s, w2t, w2r, w2a, b_et2s0.reshape(1, H),
        W_et2s1.T.astype(bf16), b_et2s1.reshape(1, H),
    )

    zeros = jnp.zeros((NPT, H), f32)
    etgt3 = edge_list[1].reshape(NS, SNCH, SC_C)
    esrc3 = edge_list[0].reshape(NS, SNCH, SC_C)
    agg1, agg2 = _scatter_k(h1, h2, etgt3, esrc3, zeros)

    tgt_out, src_out = _node_call(
        tgt_node_feat, agg1, src_node_feat, agg2,
        W_nt0[:, :D].T.astype(bf16), W_nt0[:, D:].T.astype(bf16),
        b_nt0.reshape(1, H), W_nt1.T.astype(bf16), b_nt1.reshape(1, H),
        W_ns0[:, :D].T.astype(bf16), W_ns0[:, D:].T.astype(bf16),
        b_ns0.reshape(1, H), W_ns1.T.astype(bf16), b_ns1.reshape(1, H),
    )
    return (tgt_out, src_out)


# speedup vs baseline: 4.1948x; 1.5689x over previous
import functools

import jax
import jax.numpy as jnp
from jax import lax
from jax.experimental import pallas as pl
from jax.experimental.pallas import tpu as pltpu
from jax.experimental.pallas import tpu_sc as plsc

N = 10000
E = 320000
D = 128
H = 128
EA = 16
CW = 16

NC = 2
NS = 16
NW = NC * NS

_sc_mesh = plsc.VectorSubcoreMesh(core_axis_name="c", subcore_axis_name="s")
_sc_params = pltpu.CompilerParams(use_tc_tiling_on_sc=False)

EPW = E // NW
GC = 40
GNCH = EPW // GC
GR = 5
GNG = GNCH // GR


@functools.partial(
    pl.kernel,
    out_type=(
        jax.ShapeDtypeStruct((E, D), jnp.float32),
        jax.ShapeDtypeStruct((E, D), jnp.float32),
        jax.ShapeDtypeStruct((E, CW), jnp.float32),
        jax.ShapeDtypeStruct((E, CW), jnp.float32),
    ),
    mesh=_sc_mesh,
    scratch_types=[
        pltpu.VMEM((2, EPW), jnp.int32),
        [pltpu.VMEM((GC, D), jnp.float32) for _ in range(GR)],
        [pltpu.VMEM((GC, D), jnp.float32) for _ in range(GR)],
        [pltpu.VMEM((GC, CW), jnp.float32) for _ in range(GR)],
        [pltpu.VMEM((GC, CW), jnp.float32) for _ in range(GR)],
        [pltpu.SemaphoreType.DMA for _ in range(GR)],
        [pltpu.SemaphoreType.DMA for _ in range(GR)],
    ],
    compiler_params=_sc_params,
)
def _gather_k(tsrc_hbm, ttgt_hbm, csrc_hbm, ctgt_hbm, elist_hbm,
              gsf_hbm, gtf_hbm, gsc_hbm, gtc_hbm,
              idx_all, sfeat, tfeat, scrd, tcrd, gsems, wsems):
    c = lax.axis_index("c")
    s = lax.axis_index("s")
    wid = s * NC + c
    base = pl.multiple_of(wid * EPW, 8)
    pltpu.sync_copy(elist_hbm.at[:, pl.ds(base, EPW)], idx_all)

    def pairs(b):
        return ((tsrc_hbm, sfeat[b], 0), (ttgt_hbm, tfeat[b], 1),
                (csrc_hbm, scrd[b], 0), (ctgt_hbm, tcrd[b], 1))

    def start_gathers(b, cof):
        for tab, buf, which in pairs(b):
            idx = idx_all.at[which, pl.ds(cof, GC)]
            pltpu.async_copy(tab.at[idx], buf, gsems[b])

    def wait_gathers(b, cof):
        for tab, buf, which in pairs(b):
            idx = idx_all.at[which, pl.ds(cof, GC)]
            pltpu.make_async_copy(tab.at[idx], buf, gsems[b]).wait()

    def outs(b, goff):
        return ((sfeat[b], gsf_hbm), (tfeat[b], gtf_hbm),
                (scrd[b], gsc_hbm), (tcrd[b], gtc_hbm))

    for b in range(GR):
        start_gathers(b, b * GC)

    def body(g, carry):
        wdescs = []
        for b in range(GR):
            cof = pl.multiple_of(g * (GR * GC) + b * GC, 8)
            goff = pl.multiple_of(base + cof, 8)
            wait_gathers(b, cof)
            slot = []
            for buf, out in outs(b, goff):
                slot.append(pltpu.async_copy(buf, out.at[pl.ds(goff, GC)], wsems[b]))
            wdescs.append(slot)
        for b in range(GR):
            for d in wdescs[b]:
                d.wait()

            @pl.when(g < GNG - 1)
            def _(b=b):
                ncof = pl.multiple_of((g + 1) * (GR * GC) + b * GC, 8)
                start_gathers(b, ncof)
        return carry

    lax.fori_loop(0, GNG, body, 0)


EB = 2560


def _edge_body(gsf, gtf, gsc, gtc, eat,
               w1s, w1t, w1r, w1a, b10, w11, b11,
               w2s, w2t, w2r, w2a, b20, w21, b21,
               h1o, h2o):
    dd = gtc[...] - gsc[...]
    radial = jnp.sum(dd * dd, axis=1, keepdims=True)
    src = gsf[...].astype(jnp.bfloat16)
    tgtf = gtf[...].astype(jnp.bfloat16)
    eab = eat[...].astype(jnp.bfloat16)

    def mlp(ws, wt, wr, wa, b0, w1, b1):
        u = jnp.dot(src, ws[...], preferred_element_type=jnp.float32)
        u = u + jnp.dot(tgtf, wt[...], preferred_element_type=jnp.float32)
        u = u + lax.dot_general(eab, wa[...], (((0,), (0,)), ((), ())),
                            preferred_element_type=jnp.float32)
        u = u + radial * wr[...]
        u = u + b0[...]
        z = jnp.maximum(u, 0.0).astype(jnp.bfloat16)
        h = jnp.dot(z, w1[...], preferred_element_type=jnp.float32) + b1[...]
        return jnp.maximum(h, 0.0)

    h1o[...] = mlp(w1s, w1t, w1r, w1a, b10, w11, b11)
    h2o[...] = mlp(w2s, w2t, w2r, w2a, b20, w21, b21)


def _full(shape):
    return pl.BlockSpec(shape, lambda i: (0, 0))


_edge_call = pl.pallas_call(
    _edge_body,
    grid=(E // EB,),
    in_specs=[
        pl.BlockSpec((EB, D), lambda i: (i, 0)),
        pl.BlockSpec((EB, D), lambda i: (i, 0)),
        pl.BlockSpec((EB, CW), lambda i: (i, 0)),
        pl.BlockSpec((EB, CW), lambda i: (i, 0)),
        pl.BlockSpec((EA, EB), lambda i: (0, i)),
        _full((D, H)), _full((D, H)), _full((1, H)), _full((EA, H)),
        _full((1, H)), _full((H, H)), _full((1, H)),
        _full((D, H)), _full((D, H)), _full((1, H)), _full((EA, H)),
        _full((1, H)), _full((H, H)), _full((1, H)),
    ],
    out_specs=[
        pl.BlockSpec((EB, H), lambda i: (i, 0)),
        pl.BlockSpec((EB, H), lambda i: (i, 0)),
    ],
    out_shape=[
        jax.ShapeDtypeStruct((E, H), jnp.float32),
        jax.ShapeDtypeStruct((E, H), jnp.float32),
    ],
)


EPT = E // NS
SC_C = 80
SNCH = EPT // SC_C
SR = 2
SNG = SNCH // SR
NPT = N // NS


@functools.partial(
    pl.kernel,
    out_type=(
        jax.ShapeDtypeStruct((N, H), jnp.float32),
        jax.ShapeDtypeStruct((N, H), jnp.float32),
    ),
    mesh=_sc_mesh,
    scratch_types=[
        pltpu.VMEM((SNCH, SC_C), jnp.int32),
        [pltpu.VMEM((SC_C, H), jnp.float32) for _ in range(SR)],
        pltpu.VMEM_SHARED((N, H), jnp.float32),
        [pltpu.SemaphoreType.DMA for _ in range(SR)],
        [pltpu.SemaphoreType.DMA for _ in range(SR)],
    ],
    compiler_params=_sc_params,
)
def _scatter_k(h1_hbm, h2_hbm, etgt_hbm, esrc_hbm, zeros_hbm,
               agg1_hbm, agg2_hbm, idxm, rows, acc_sh, lsems, ssems):
    c = lax.axis_index("c")
    s = lax.axis_index("s")
    nbase = pl.multiple_of(s * NPT, 8)
    pltpu.sync_copy(zeros_hbm, acc_sh.at[pl.ds(nbase, NPT)])

    @pl.when(c == 0)
    def _():
        pltpu.sync_copy(etgt_hbm.at[s], idxm)

    @pl.when(c == 1)
    def _():
        pltpu.sync_copy(esrc_hbm.at[s], idxm)

    plsc.subcore_barrier()

    def run(h_hbm):
        base = pl.multiple_of(s * EPT, 8)

        def start_load(b, j):
            off = pl.multiple_of(base + j * SC_C, 8)
            pltpu.async_copy(h_hbm.at[pl.ds(off, SC_C)], rows[b], lsems[b])

        for b in range(SR):
            start_load(b, b)

        def body(g, carry):
            sdescs = []
            for b in range(SR):
                j = g * SR + b
                off = pl.multiple_of(base + j * SC_C, 8)
                pltpu.make_async_copy(
                    h_hbm.at[pl.ds(off, SC_C)], rows[b], lsems[b]).wait()
                sdescs.append(pltpu.async_copy(
                    rows[b], acc_sh.at[idxm.at[j]], ssems[b], add=True))
            for b in range(SR):
                sdescs[b].wait()

                @pl.when(g < SNG - 1)
                def _(b=b):
                    start_load(b, (g + 1) * SR + b)
            return carry

        lax.fori_loop(0, SNG, body, 0)

    @pl.when(c == 0)
    def _():
        run(h1_hbm)

    @pl.when(c == 1)
    def _():
        run(h2_hbm)

    plsc.subcore_barrier()

    @pl.when(c == 0)
    def _():
        pltpu.sync_copy(acc_sh.at[pl.ds(nbase, NPT)],
                        agg1_hbm.at[pl.ds(nbase, NPT)])

    @pl.when(c == 1)
    def _():
        pltpu.sync_copy(acc_sh.at[pl.ds(nbase, NPT)],
                        agg2_hbm.at[pl.ds(nbase, NPT)])


NB = 2000


def _node_body(tf, a1, sf, a2,
               wtf, wta, bt0, wt1, bt1,
               wsf, wsa, bs0, ws1, bs1,
               tgt_o, src_o):
    def upd(x, a, wf, wa, b0, w1, b1):
        xb = x.astype(jnp.bfloat16)
        ab = a.astype(jnp.bfloat16)
        u = jnp.dot(xb, wf[...], preferred_element_type=jnp.float32)
        u = u + jnp.dot(ab, wa[...], preferred_element_type=jnp.float32)
        u = u + b0[...]
        z = jnp.maximum(u, 0.0).astype(jnp.bfloat16)
        return x + jnp.dot(z, w1[...], preferred_element_type=jnp.float32) + b1[...]

    tgt_o[...] = upd(tf[...], a1[...], wtf, wta, bt0, wt1, bt1)
    src_o[...] = upd(sf[...], a2[...], wsf, wsa, bs0, ws1, bs1)


_node_call = pl.pallas_call(
    _node_body,
    grid=(N // NB,),
    in_specs=[
        pl.BlockSpec((NB, D), lambda i: (i, 0)),
        pl.BlockSpec((NB, H), lambda i: (i, 0)),
        pl.BlockSpec((NB, D), lambda i: (i, 0)),
        pl.BlockSpec((NB, H), lambda i: (i, 0)),
        _full((D, H)), _full((H, H)), _full((1, H)), _full((H, H)), _full((1, H)),
        _full((D, H)), _full((H, H)), _full((1, H)), _full((H, H)), _full((1, H)),
    ],
    out_specs=[
        pl.BlockSpec((NB, D), lambda i: (i, 0)),
        pl.BlockSpec((NB, D), lambda i: (i, 0)),
    ],
    out_shape=[
        jax.ShapeDtypeStruct((N, D), jnp.float32),
        jax.ShapeDtypeStruct((N, D), jnp.float32),
    ],
)


def kernel(src_node_feat, tgt_node_feat, src_node_coord, tgt_node_coord,
           edge_list, edge_attr,
           W_es2t0, b_es2t0, W_es2t1, b_es2t1,
           W_et2s0, b_et2s0, W_et2s1, b_et2s1,
           W_nt0, b_nt0, W_nt1, b_nt1,
           W_ns0, b_ns0, W_ns1, b_ns1):
    f32 = jnp.float32
    bf16 = jnp.bfloat16

    tsrc = src_node_feat
    ttgt = tgt_node_feat
    csrc = jnp.pad(src_node_coord, ((0, 0), (0, CW - 3)))
    ctgt = jnp.pad(tgt_node_coord, ((0, 0), (0, CW - 3)))

    gsf, gtf, gsc, gtc = _gather_k(tsrc, ttgt, csrc, ctgt, edge_list)

    def esplit(W):
        return (W[:, :D].T.astype(bf16), W[:, D:2 * D].T.astype(bf16),
                W[:, 2 * D].reshape(1, H), W[:, 2 * D + 1:].T.astype(bf16))

    w1s, w1t, w1r, w1a = esplit(W_es2t0)
    w2s, w2t, w2r, w2a = esplit(W_et2s0)

    h1, h2 = _edge_call(
        gsf, gtf, gsc, gtc, edge_attr.T,
        w1s, w1t, w1r, w1a, b_es2t0.reshape(1, H),
        W_es2t1.T.astype(bf16), b_es2t1.reshape(1, H),
        w2s, w2t, w2r, w2a, b_et2s0.reshape(1, H),
        W_et2s1.T.astype(bf16), b_et2s1.reshape(1, H),
    )

    zeros = jnp.zeros((NPT, H), f32)
    etgt3 = edge_list[1].reshape(NS, SNCH, SC_C)
    esrc3 = edge_list[0].reshape(NS, SNCH, SC_C)
    agg1, agg2 = _scatter_k(h1, h2, etgt3, esrc3, zeros)

    tgt_out, src_out = _node_call(
        tgt_node_feat, agg1, src_node_feat, agg2,
        W_nt0[:, :D].T.astype(bf16), W_nt0[:, D:].T.astype(bf16),
        b_nt0.reshape(1, H), W_nt1.T.astype(bf16), b_nt1.reshape(1, H),
        W_ns0[:, :D].T.astype(bf16), W_ns0[:, D:].T.astype(bf16),
        b_ns0.reshape(1, H), W_ns1.T.astype(bf16), b_ns1.reshape(1, H),
    )
    return (tgt_out, src_out)


# speedup vs baseline: 4.8370x; 1.1531x over previous
import functools

import jax
import jax.numpy as jnp
from jax import lax
from jax.experimental import pallas as pl
from jax.experimental.pallas import tpu as pltpu
from jax.experimental.pallas import tpu_sc as plsc

N = 10000
E = 320000
D = 128
H = 128
EA = 16
CW = 16

NSEG = 2
ES = E // NSEG

NC = 2
NS = 16
NW = NC * NS

_sc_mesh = plsc.VectorSubcoreMesh(core_axis_name="c", subcore_axis_name="s")
_sc_params = pltpu.CompilerParams(use_tc_tiling_on_sc=False)

EPW = ES // NW
GC = 40
GNCH = EPW // GC
GR = 5
GNG = GNCH // GR


@functools.partial(
    pl.kernel,
    out_type=(
        jax.ShapeDtypeStruct((ES, D), jnp.float32),
        jax.ShapeDtypeStruct((ES, D), jnp.float32),
        jax.ShapeDtypeStruct((ES, CW), jnp.float32),
        jax.ShapeDtypeStruct((ES, CW), jnp.float32),
    ),
    mesh=_sc_mesh,
    scratch_types=[
        pltpu.VMEM((2, EPW), jnp.int32),
        [pltpu.VMEM((GC, D), jnp.float32) for _ in range(GR)],
        [pltpu.VMEM((GC, D), jnp.float32) for _ in range(GR)],
        [pltpu.VMEM((GC, CW), jnp.float32) for _ in range(GR)],
        [pltpu.VMEM((GC, CW), jnp.float32) for _ in range(GR)],
        [pltpu.SemaphoreType.DMA for _ in range(GR)],
        [pltpu.SemaphoreType.DMA for _ in range(GR)],
    ],
    compiler_params=_sc_params,
)
def _gather_k(tsrc_hbm, ttgt_hbm, csrc_hbm, ctgt_hbm, elist_hbm,
              gsf_hbm, gtf_hbm, gsc_hbm, gtc_hbm,
              idx_all, sfeat, tfeat, scrd, tcrd, gsems, wsems):
    c = lax.axis_index("c")
    s = lax.axis_index("s")
    wid = s * NC + c
    base = pl.multiple_of(wid * EPW, 8)
    pltpu.sync_copy(elist_hbm.at[:, pl.ds(base, EPW)], idx_all)

    def pairs(b):
        return ((tsrc_hbm, sfeat[b], 0), (ttgt_hbm, tfeat[b], 1),
                (csrc_hbm, scrd[b], 0), (ctgt_hbm, tcrd[b], 1))

    def start_gathers(b, cof):
        for tab, buf, which in pairs(b):
            idx = idx_all.at[which, pl.ds(cof, GC)]
            pltpu.async_copy(tab.at[idx], buf, gsems[b])

    def wait_gathers(b, cof):
        for tab, buf, which in pairs(b):
            idx = idx_all.at[which, pl.ds(cof, GC)]
            pltpu.make_async_copy(tab.at[idx], buf, gsems[b]).wait()

    def outs(b):
        return ((sfeat[b], gsf_hbm), (tfeat[b], gtf_hbm),
                (scrd[b], gsc_hbm), (tcrd[b], gtc_hbm))

    for b in range(GR):
        start_gathers(b, b * GC)

    def body(g, carry):
        wdescs = []
        for b in range(GR):
            cof = pl.multiple_of(g * (GR * GC) + b * GC, 8)
            goff = pl.multiple_of(base + cof, 8)
            wait_gathers(b, cof)
            slot = []
            for buf, out in outs(b):
                slot.append(pltpu.async_copy(buf, out.at[pl.ds(goff, GC)], wsems[b]))
            wdescs.append(slot)
        for b in range(GR):
            for d in wdescs[b]:
                d.wait()

            @pl.when(g < GNG - 1)
            def _(b=b):
                ncof = pl.multiple_of((g + 1) * (GR * GC) + b * GC, 8)
                start_gathers(b, ncof)
        return carry

    lax.fori_loop(0, GNG, body, 0)


EB = 3200


def _edge_body(gsf, gtf, gsc, gtc, eat,
               w1s, w1t, w1r, w1a, b10, w11, b11,
               w2s, w2t, w2r, w2a, b20, w21, b21,
               h1o, h2o):
    dd = gtc[...] - gsc[...]
    radial = jnp.sum(dd * dd, axis=1, keepdims=True)
    src = gsf[...].astype(jnp.bfloat16)
    tgtf = gtf[...].astype(jnp.bfloat16)
    eab = eat[...].astype(jnp.bfloat16)

    def mlp(ws, wt, wr, wa, b0, w1, b1):
        u = jnp.dot(src, ws[...], preferred_element_type=jnp.float32)
        u = u + jnp.dot(tgtf, wt[...], preferred_element_type=jnp.float32)
        u = u + lax.dot_general(eab, wa[...], (((0,), (0,)), ((), ())),
                            preferred_element_type=jnp.float32)
        u = u + radial * wr[...]
        u = u + b0[...]
        z = jnp.maximum(u, 0.0).astype(jnp.bfloat16)
        h = jnp.dot(z, w1[...], preferred_element_type=jnp.float32) + b1[...]
        return jnp.maximum(h, 0.0)

    h1o[...] = mlp(w1s, w1t, w1r, w1a, b10, w11, b11)
    h2o[...] = mlp(w2s, w2t, w2r, w2a, b20, w21, b21)


def _full(shape):
    return pl.BlockSpec(shape, lambda i: (0, 0))


_edge_call = pl.pallas_call(
    _edge_body,
    grid=(ES // EB,),
    in_specs=[
        pl.BlockSpec((EB, D), lambda i: (i, 0)),
        pl.BlockSpec((EB, D), lambda i: (i, 0)),
        pl.BlockSpec((EB, CW), lambda i: (i, 0)),
        pl.BlockSpec((EB, CW), lambda i: (i, 0)),
        pl.BlockSpec((EA, EB), lambda i: (0, i)),
        _full((D, H)), _full((D, H)), _full((1, H)), _full((EA, H)),
        _full((1, H)), _full((H, H)), _full((1, H)),
        _full((D, H)), _full((D, H)), _full((1, H)), _full((EA, H)),
        _full((1, H)), _full((H, H)), _full((1, H)),
    ],
    out_specs=[
        pl.BlockSpec((EB, H), lambda i: (i, 0)),
        pl.BlockSpec((EB, H), lambda i: (i, 0)),
    ],
    out_shape=[
        jax.ShapeDtypeStruct((ES, H), jnp.float32),
        jax.ShapeDtypeStruct((ES, H), jnp.float32),
    ],
)


EPT = ES // NS
SC_C = 40
SNCH = EPT // SC_C
SR = 5
SNG = SNCH // SR
NPT = N // NS


@functools.partial(
    pl.kernel,
    out_type=(
        jax.ShapeDtypeStruct((N, H), jnp.float32),
        jax.ShapeDtypeStruct((N, H), jnp.float32),
    ),
    mesh=_sc_mesh,
    scratch_types=[
        pltpu.VMEM((SNCH, SC_C), jnp.int32),
        [pltpu.VMEM((SC_C, H), jnp.float32) for _ in range(SR)],
        pltpu.VMEM_SHARED((N, H), jnp.float32),
        [pltpu.SemaphoreType.DMA for _ in range(SR)],
        [pltpu.SemaphoreType.DMA for _ in range(SR)],
    ],
    compiler_params=_sc_params,
)
def _scatter_k(h1_hbm, h2_hbm, etgt_hbm, esrc_hbm, init1_hbm, init2_hbm,
               agg1_hbm, agg2_hbm, idxm, rows, acc_sh, lsems, ssems):
    c = lax.axis_index("c")
    s = lax.axis_index("s")
    nbase = pl.multiple_of(s * NPT, 8)

    @pl.when(c == 0)
    def _():
        pltpu.sync_copy(init1_hbm.at[pl.ds(nbase, NPT)],
                        acc_sh.at[pl.ds(nbase, NPT)])
        pltpu.sync_copy(etgt_hbm.at[s], idxm)

    @pl.when(c == 1)
    def _():
        pltpu.sync_copy(init2_hbm.at[pl.ds(nbase, NPT)],
                        acc_sh.at[pl.ds(nbase, NPT)])
        pltpu.sync_copy(esrc_hbm.at[s], idxm)

    plsc.subcore_barrier()

    def run(h_hbm):
        base = pl.multiple_of(s * EPT, 8)

        def start_load(b, j):
            off = pl.multiple_of(base + j * SC_C, 8)
            pltpu.async_copy(h_hbm.at[pl.ds(off, SC_C)], rows[b], lsems[b])

        for b in range(SR):
            start_load(b, b)

        def body(g, carry):
            sdescs = []
            for b in range(SR):
                j = g * SR + b
                off = pl.multiple_of(base + j * SC_C, 8)
                pltpu.make_async_copy(
                    h_hbm.at[pl.ds(off, SC_C)], rows[b], lsems[b]).wait()
                sdescs.append(pltpu.async_copy(
                    rows[b], acc_sh.at[idxm.at[j]], ssems[b], add=True))
            for b in range(SR):
                sdescs[b].wait()

                @pl.when(g < SNG - 1)
                def _(b=b):
                    start_load(b, (g + 1) * SR + b)
            return carry

        lax.fori_loop(0, SNG, body, 0)

    @pl.when(c == 0)
    def _():
        run(h1_hbm)

    @pl.when(c == 1)
    def _():
        run(h2_hbm)

    plsc.subcore_barrier()

    @pl.when(c == 0)
    def _():
        pltpu.sync_copy(acc_sh.at[pl.ds(nbase, NPT)],
                        agg1_hbm.at[pl.ds(nbase, NPT)])

    @pl.when(c == 1)
    def _():
        pltpu.sync_copy(acc_sh.at[pl.ds(nbase, NPT)],
                        agg2_hbm.at[pl.ds(nbase, NPT)])


NB = 2000


def _node_body(tf, a1, sf, a2,
               wtf, wta, bt0, wt1, bt1,
               wsf, wsa, bs0, ws1, bs1,
               tgt_o, src_o):
    def upd(x, a, wf, wa, b0, w1, b1):
        xb = x.astype(jnp.bfloat16)
        ab = a.astype(jnp.bfloat16)
        u = jnp.dot(xb, wf[...], preferred_element_type=jnp.float32)
        u = u + jnp.dot(ab, wa[...], preferred_element_type=jnp.float32)
        u = u + b0[...]
        z = jnp.maximum(u, 0.0).astype(jnp.bfloat16)
        return x + jnp.dot(z, w1[...], preferred_element_type=jnp.float32) + b1[...]

    tgt_o[...] = upd(tf[...], a1[...], wtf, wta, bt0, wt1, bt1)
    src_o[...] = upd(sf[...], a2[...], wsf, wsa, bs0, ws1, bs1)


_node_call = pl.pallas_call(
    _node_body,
    grid=(N // NB,),
    in_specs=[
        pl.BlockSpec((NB, D), lambda i: (i, 0)),
        pl.BlockSpec((NB, H), lambda i: (i, 0)),
        pl.BlockSpec((NB, D), lambda i: (i, 0)),
        pl.BlockSpec((NB, H), lambda i: (i, 0)),
        _full((D, H)), _full((H, H)), _full((1, H)), _full((H, H)), _full((1, H)),
        _full((D, H)), _full((H, H)), _full((1, H)), _full((H, H)), _full((1, H)),
    ],
    out_specs=[
        pl.BlockSpec((NB, D), lambda i: (i, 0)),
        pl.BlockSpec((NB, D), lambda i: (i, 0)),
    ],
    out_shape=[
        jax.ShapeDtypeStruct((N, D), jnp.float32),
        jax.ShapeDtypeStruct((N, D), jnp.float32),
    ],
)


def kernel(src_node_feat, tgt_node_feat, src_node_coord, tgt_node_coord,
           edge_list, edge_attr,
           W_es2t0, b_es2t0, W_es2t1, b_es2t1,
           W_et2s0, b_et2s0, W_et2s1, b_et2s1,
           W_nt0, b_nt0, W_nt1, b_nt1,
           W_ns0, b_ns0, W_ns1, b_ns1):
    f32 = jnp.float32
    bf16 = jnp.bfloat16

    csrc = jnp.pad(src_node_coord, ((0, 0), (0, CW - 3)))
    ctgt = jnp.pad(tgt_node_coord, ((0, 0), (0, CW - 3)))

    def esplit(W):
        return (W[:, :D].T.astype(bf16), W[:, D:2 * D].T.astype(bf16),
                W[:, 2 * D].reshape(1, H), W[:, 2 * D + 1:].T.astype(bf16))

    w1s, w1t, w1r, w1a = esplit(W_es2t0)
    w2s, w2t, w2r, w2a = esplit(W_et2s0)
    eat_full = edge_attr.T

    hs = []
    for seg in range(NSEG):
        el = lax.slice(edge_list, (0, seg * ES), (2, (seg + 1) * ES))
        gsf, gtf, gsc, gtc = _gather_k(src_node_feat, tgt_node_feat,
                                       csrc, ctgt, el)
        eat = lax.slice(eat_full, (0, seg * ES), (EA, (seg + 1) * ES))
        h1, h2 = _edge_call(
            gsf, gtf, gsc, gtc, eat,
            w1s, w1t, w1r, w1a, b_es2t0.reshape(1, H),
            W_es2t1.T.astype(bf16), b_es2t1.reshape(1, H),
            w2s, w2t, w2r, w2a, b_et2s0.reshape(1, H),
            W_et2s1.T.astype(bf16), b_et2s1.reshape(1, H),
        )
        hs.append((h1, h2))

    agg1 = jnp.zeros((N, H), f32)
    agg2 = jnp.zeros((N, H), f32)
    for seg in range(NSEG):
        h1, h2 = hs[seg]
        etgt3 = lax.slice(edge_list[1], (seg * ES,), ((seg + 1) * ES,)).reshape(
            NS, SNCH, SC_C)
        esrc3 = lax.slice(edge_list[0], (seg * ES,), ((seg + 1) * ES,)).reshape(
            NS, SNCH, SC_C)
        agg1, agg2 = _scatter_k(h1, h2, etgt3, esrc3, agg1, agg2)

    tgt_out, src_out = _node_call(
        tgt_node_feat, agg1, src_node_feat, agg2,
        W_nt0[:, :D].T.astype(bf16), W_nt0[:, D:].T.astype(bf16),
        b_nt0.reshape(1, H), W_nt1.T.astype(bf16), b_nt1.reshape(1, H),
        W_ns0[:, :D].T.astype(bf16), W_ns0[:, D:].T.astype(bf16),
        b_ns0.reshape(1, H), W_ns1.T.astype(bf16), b_ns1.reshape(1, H),
    )
    return (tgt_out, src_out)
